# baseline (speedup 1.0000x reference)
import functools

import jax
import jax.numpy as jnp
from jax import lax
from jax.experimental import pallas as pl
from jax.experimental.pallas import tpu as pltpu
from jax.experimental.pallas import tpu_sc as plsc

D = 128
CH = 128
NBUF = 5
SUP = NBUF * CH


@functools.lru_cache(maxsize=None)
def _make_gather(n_rows: int):
    info = plsc.get_sparse_core_info()
    nc, ns = info.num_cores, info.num_subcores
    nw = nc * ns
    assert n_rows % (nw * SUP) == 0
    per_w = n_rows // nw
    n_super = per_w // SUP

    mesh = plsc.VectorSubcoreMesh(core_axis_name="c", subcore_axis_name="s")

    @functools.partial(
        pl.kernel,
        out_type=jax.ShapeDtypeStruct((n_rows, D), jnp.float32),
        mesh=mesh,
        scratch_types=[
            pltpu.VMEM((per_w,), jnp.int32),
            pltpu.VMEM((NBUF, CH, D), jnp.float32),
            pltpu.SemaphoreType.DMA((NBUF,)),
            pltpu.SemaphoreType.DMA((NBUF,)),
        ],
    )
    def k(tab_hbm, idx_hbm, out_hbm, idx_v, rows, gsem, ssem):
        wid = lax.axis_index("s") * nc + lax.axis_index("c")
        base = wid * per_w

        pltpu.sync_copy(idx_hbm.at[pl.ds(base, per_w)], idx_v)

        for b in range(NBUF):
            pltpu.async_copy(
                tab_hbm.at[idx_v.at[pl.ds(b * CH, CH)]], rows.at[b], gsem.at[b]
            )

        def sup(s, carry):
            for b in range(NBUF):
                pltpu.make_async_copy(
                    tab_hbm.at[pl.ds(0, CH)], rows.at[b], gsem.at[b]
                ).wait()
                pltpu.async_copy(
                    rows.at[b],
                    out_hbm.at[pl.ds(base + s * SUP + b * CH, CH)],
                    ssem.at[b],
                )
            for b in range(NBUF):
                pltpu.make_async_copy(
                    rows.at[b], out_hbm.at[pl.ds(0, CH)], ssem.at[b]
                ).wait()
                pltpu.async_copy(
                    tab_hbm.at[idx_v.at[pl.ds((s + 1) * SUP + b * CH, CH)]],
                    rows.at[b],
                    gsem.at[b],
                )
            return carry

        lax.fori_loop(0, n_super - 1, sup, 0)

        last = base + (n_super - 1) * SUP
        for b in range(NBUF):
            pltpu.make_async_copy(
                tab_hbm.at[pl.ds(0, CH)], rows.at[b], gsem.at[b]
            ).wait()
            pltpu.async_copy(
                rows.at[b], out_hbm.at[pl.ds(last + b * CH, CH)], ssem.at[b]
            )
        for b in range(NBUF):
            pltpu.make_async_copy(
                rows.at[b], out_hbm.at[pl.ds(0, CH)], ssem.at[b]
            ).wait()

    return k


def kernel(indices, pe):
    b, kk = indices.shape
    table = pe[0]
    idx = indices.reshape(-1).astype(jnp.int32)
    out = _make_gather(b * kk)(table, idx)
    return out.reshape(b, kk, D)


# speedup vs baseline: 1.2056x; 1.2056x over previous
import functools
import jax, jax.numpy as jnp
from jax import lax
from jax.experimental import pallas as pl
from jax.experimental.pallas import tpu as pltpu
from jax.experimental.pallas import tpu_sc as plsc

D = 128
CH = 128
NBUF = 2
SUP = NBUF * CH

@functools.lru_cache(maxsize=None)
def _make_gather(n_rows: int, n_tab: int):
    info = plsc.get_sparse_core_info()
    nc, ns = info.num_cores, info.num_subcores
    nw = nc * ns
    per_w = n_rows // nw
    n_super = per_w // SUP
    tab_per_s = n_tab // ns
    mesh = plsc.VectorSubcoreMesh(core_axis_name="c", subcore_axis_name="s")

    @functools.partial(
        pl.kernel,
        out_type=jax.ShapeDtypeStruct((n_rows, D), jnp.float32),
        mesh=mesh,
        scratch_types=[
            pltpu.VMEM((2, SUP), jnp.int32),
            pltpu.VMEM((NBUF, CH, D), jnp.float32),
            pltpu.VMEM_SHARED((8192, D), jnp.float32),
            pltpu.SemaphoreType.DMA((NBUF,)),
            pltpu.SemaphoreType.DMA((NBUF,)),
        ],
    )
    def k(tab_hbm, idx_hbm, out_hbm, idx_v, rows, stab, gsem, ssem):
        cid = lax.axis_index("c")
        sid = lax.axis_index("s")
        wid = sid * nc + cid
        base = wid * per_w
        pltpu.sync_copy(
            tab_hbm.at[pl.ds(sid * tab_per_s, tab_per_s)],
            stab.at[pl.ds(sid * tab_per_s, tab_per_s)],
        )
        pltpu.sync_copy(idx_hbm.at[pl.ds(base, SUP)], idx_v.at[0])
        plsc.subcore_barrier()
        for b in range(NBUF):
            pltpu.async_copy(
                stab.at[idx_v.at[0, pl.ds(b * CH, CH)]], rows.at[b], gsem.at[b]
            )
        def sup(s, carry):
            nxt = (s + 1) % 2
            pltpu.sync_copy(
                idx_hbm.at[pl.ds(base + (s + 1) * SUP, SUP)], idx_v.at[nxt]
            )
            for b in range(NBUF):
                pltpu.make_async_copy(
                    stab.at[pl.ds(0, CH)], rows.at[b], gsem.at[b]
                ).wait()
                pltpu.async_copy(
                    rows.at[b],
                    out_hbm.at[pl.ds(base + s * SUP + b * CH, CH)],
                    ssem.at[b],
                )
            for b in range(NBUF):
                pltpu.make_async_copy(
                    rows.at[b], out_hbm.at[pl.ds(0, CH)], ssem.at[b]
                ).wait()
                pltpu.async_copy(
                    stab.at[idx_v.at[nxt, pl.ds(b * CH, CH)]],
                    rows.at[b],
                    gsem.at[b],
                )
            return carry
        lax.fori_loop(0, n_super - 1, sup, 0)
        last = base + (n_super - 1) * SUP
        for b in range(NBUF):
            pltpu.make_async_copy(
                stab.at[pl.ds(0, CH)], rows.at[b], gsem.at[b]
            ).wait()
            pltpu.async_copy(
                rows.at[b], out_hbm.at[pl.ds(last + b * CH, CH)], ssem.at[b]
            )
        for b in range(NBUF):
            pltpu.make_async_copy(
                rows.at[b], out_hbm.at[pl.ds(0, CH)], ssem.at[b]
            ).wait()
    return k

def kernel(indices, pe):
    b, kk = indices.shape
    table = pe[0]
    idx = indices.reshape(-1).astype(jnp.int32)
    out = _make_gather(b * kk, table.shape[0])(table, idx)
    return out.reshape(b, kk, D)
